# baseline (speedup 1.0000x reference)
import functools

import jax
import jax.numpy as jnp
from jax import lax
from jax.experimental import pallas as pl
from jax.experimental.pallas import tpu as pltpu
from jax.experimental.pallas import tpu_sc as plsc

N_NODES = 10000
NP = 10240
N_EDGES = 320000
CW = 64
NCHUNK = 8
NCORE = 2
NTILE = 16
EPT = 20480
E_PAD = EPT * NTILE
WIN = 128
NBUF = 8
STG = 5120
NITER = STG // (WIN * NBUF)
NSTG = EPT // STG
RPT = NP // NTILE
IBK = 128


def _make_spmm():
    mesh = plsc.VectorSubcoreMesh(core_axis_name="c", subcore_axis_name="s")

    scratch = (
        [pltpu.VMEM_SHARED((NP, CW), jnp.float32)]
        + [pltpu.VMEM((STG,), jnp.int32) for _ in range(2)]
        + [pltpu.VMEM((STG,), jnp.float32)]
        + [pltpu.VMEM((WIN, CW), jnp.float32) for _ in range(NBUF)]
        + [pltpu.VMEM((WIN,), jnp.int32) for _ in range(NBUF)]
        + [pltpu.VMEM((WIN,), jnp.int32) for _ in range(NBUF)]
        + [pltpu.SemaphoreType.DMA for _ in range(2 * NBUF)]
    )

    @functools.partial(
        pl.kernel,
        out_type=jax.ShapeDtypeStruct((NCHUNK * NP, CW), jnp.float32),
        mesh=mesh,
        scratch_types=scratch,
        compiler_params=pltpu.CompilerParams(use_tc_tiling_on_sc=False),
    )
    def spmm(*refs):
        row_hbm, col_hbm, val_hbm, src_hbm = refs[:4]
        out_hbm = refs[4]
        sc = refs[5:]
        acc = sc[0]
        rowbig, colbig, valbig = sc[1], sc[2], sc[3]
        gath = sc[4:4 + NBUF]
        coltmp = sc[4 + NBUF:4 + 2 * NBUF]
        rowtmp = sc[4 + 2 * NBUF:4 + 3 * NBUF]
        gsem = sc[4 + 3 * NBUF:4 + 3 * NBUF + NBUF]
        ssem = sc[4 + 3 * NBUF + NBUF:]

        cid = lax.axis_index("c")
        sid = lax.axis_index("s")
        e0 = sid * EPT
        r0 = sid * RPT

        def _pass(p, _):
            chunk = NCORE * p + cid
            coff = chunk * NP

            iblk = gath[0].at[pl.ds(0, IBK)]
            zv = jnp.zeros((16,), jnp.float32)

            def _zero(r, _):
                for u in range(CW // 16):
                    gath[0][r, pl.ds(u * 16, 16)] = zv
                return 0

            lax.fori_loop(0, IBK, _zero, 0)

            def _initz(b, _):
                pltpu.sync_copy(iblk, acc.at[pl.ds(r0 + b * IBK, IBK)])
                return 0

            lax.fori_loop(0, RPT // IBK, _initz, 0)
            plsc.subcore_barrier()

            def _win_idx(bj, w):
                o = w * WIN
                for u in range(WIN // 16):
                    s = pl.ds(u * 16, 16)
                    so = pl.ds(o + u * 16, 16)
                    coltmp[bj][s] = colbig[so] + coff
                    rowtmp[bj][s] = rowbig[so]

            def _gather(bj):
                pltpu.async_copy(src_hbm.at[coltmp[bj]], gath[bj], gsem[bj])

            def _wait_gather(bj):
                pltpu.make_async_copy(
                    src_hbm.at[coltmp[bj]], gath[bj], gsem[bj]).wait()

            def _scatter(bj):
                pltpu.async_copy(
                    gath[bj], acc.at[rowtmp[bj]], ssem[bj], add=True)

            def _wait_scatter(bj):
                pltpu.make_async_copy(
                    gath[bj], acc.at[rowtmp[bj]], ssem[bj]).wait()

            def _mul_win(bj, w):
                o = w * WIN

                @plsc.parallel_loop(0, WIN // 16, unroll=4)
                def _mul(k2):
                    vv = valbig[pl.ds(o + k2 * 16, 16)]
                    for t in range(16):
                        e = k2 * 16 + t
                        v = vv[t]
                        for u in range(CW // 16):
                            s = pl.ds(u * 16, 16)
                            gath[bj][e, s] = gath[bj][e, s] * v

            HB = NBUF // 2

            def _stage(stg, _):
                eb = e0 + stg * STG
                pltpu.sync_copy(row_hbm.at[pl.ds(eb, STG)], rowbig)
                pltpu.sync_copy(col_hbm.at[pl.ds(eb, STG)], colbig)
                pltpu.sync_copy(val_hbm.at[pl.ds(eb, STG)], valbig)

                for j in range(HB):
                    _win_idx(j, j)
                    _gather(j)

                def _step(k, _):
                    gA = 2 * k
                    gB = 2 * k + 1
                    for j in range(HB):
                        bj = HB + j

                        @pl.when(k > 0)
                        def _():
                            _wait_scatter(bj)
                        _win_idx(bj, gB * HB + j)
                        _gather(bj)
                    for j in range(HB):
                        _wait_gather(j)
                        _mul_win(j, gA * HB + j)
                        _scatter(j)
                    for j in range(HB):
                        @pl.when(k < NITER - 1)
                        def _():
                            _wait_scatter(j)
                            _win_idx(j, (2 * k + 2) * HB + j)
                            _gather(j)
                    for j in range(HB):
                        _wait_gather(HB + j)
                        _mul_win(HB + j, gB * HB + j)
                        _scatter(HB + j)
                    return 0

                lax.fori_loop(0, NITER, _step, 0)
                for j in range(NBUF):
                    _wait_scatter(j)
                return 0

            lax.fori_loop(0, NSTG, _stage, 0)
            plsc.subcore_barrier()

            pltpu.sync_copy(acc.at[pl.ds(r0, RPT)],
                            out_hbm.at[pl.ds(coff + r0, RPT)])
            plsc.subcore_barrier()
            return 0

        lax.fori_loop(0, NCHUNK // NCORE, _pass, 0)

    return spmm


_spmm = _make_spmm()

TN = 1000


def _proj_body(x0r, x1r, x2r, x3r, x4r, wr, br, outr):
    acc = jnp.zeros((TN, 512), jnp.float32)
    for m, xr in enumerate((x0r, x1r, x2r, x3r, x4r)):
        xcat = jnp.concatenate([xr[j] for j in range(NCHUNK)], axis=1)
        acc = acc + jnp.dot(xcat, wr[m], preferred_element_type=jnp.float32)
    outr[...] = acc + br[...]


def _project(xs_cm, w3, bias_t):
    in_specs = (
        [pl.BlockSpec((NCHUNK, TN, CW), lambda i: (0, i, 0)) for _ in range(5)]
        + [pl.BlockSpec((5, 512, 512), lambda i: (0, 0, 0)),
           pl.BlockSpec((1, 512), lambda i: (0, 0))]
    )
    return pl.pallas_call(
        _proj_body,
        grid=(N_NODES // TN,),
        in_specs=in_specs,
        out_specs=pl.BlockSpec((TN, 512), lambda i: (i, 0)),
        out_shape=jax.ShapeDtypeStruct((N_NODES, 512), jnp.float32),
    )(*xs_cm, w3, bias_t)


def _pad_edges(idx, val):
    pad = E_PAD - N_EDGES
    ar = jnp.arange(pad, dtype=jnp.int32)
    rows = jnp.concatenate([idx[0], ar % N_NODES])
    cols = jnp.concatenate([idx[1], (ar * 131) % N_NODES])
    vals = jnp.concatenate([val, jnp.zeros((pad,), jnp.float32)])
    return rows, cols, vals


def kernel(x, s0_idx, s0_val, s1_idx, s1_val, Theta, bias):
    b, n, c = x.shape
    x0 = x.reshape(n, c * b)
    x0c3 = jnp.pad(x0.reshape(n, NCHUNK, CW).transpose(1, 0, 2),
                   ((0, 0), (0, NP - n), (0, 0)))
    x0c = x0c3.reshape(NCHUNK * NP, CW)

    r0, c0, v0 = _pad_edges(s0_idx, s0_val)
    r1, c1, v1 = _pad_edges(s1_idx, s1_val)

    s1m = _spmm(r0, c0, v0, x0c)
    s2m = _spmm(r0, c0, v0, s1m)
    s3m = _spmm(r1, c1, v1, s1m)
    s4m = _spmm(r1, c1, v1, s3m)

    theta_r = Theta.reshape(128, 5, c)
    tm = jnp.transpose(theta_r, (1, 0, 2))
    tm_rep = jnp.repeat(tm, 4, axis=1)
    msk = jnp.tile(jnp.eye(4, dtype=jnp.float32), (128, 1))
    w2 = (tm_rep[:, :, None, :] * msk[None, :, :, None]).reshape(5, 512, 512)
    w2 = w2.at[0].add(-w2[2]).at[1].add(-w2[4])
    w2 = w2.at[2].multiply(2.0).at[4].multiply(2.0)
    bias_t = jnp.tile(bias, 4).reshape(1, 512)

    xs_cm = [a.reshape(NCHUNK, NP, CW) for a in (x0c, s1m, s2m, s3m, s4m)]
    out_cat = _project(xs_cm, w2, bias_t)
    return out_cat.reshape(n, 4, 128).transpose(1, 0, 2)


# speedup vs baseline: 1.0321x; 1.0321x over previous
import functools

import jax
import jax.numpy as jnp
from jax import lax
from jax.experimental import pallas as pl
from jax.experimental.pallas import tpu as pltpu
from jax.experimental.pallas import tpu_sc as plsc

N_NODES = 10000
NP = 10240
N_EDGES = 320000
CW = 64
NCHUNK = 8
NCORE = 2
NTILE = 16
EPT = 20480
E_PAD = EPT * NTILE
WIN = 128
NBUF = 8
STG = 5120
NITER = STG // (WIN * NBUF)
NSTG = EPT // STG
RPT = NP // NTILE
IBK = 128


def _make_spmm():
    mesh = plsc.VectorSubcoreMesh(core_axis_name="c", subcore_axis_name="s")

    scratch = (
        [pltpu.VMEM_SHARED((NP, CW), jnp.float32)]
        + [pltpu.VMEM((STG,), jnp.int32) for _ in range(2)]
        + [pltpu.VMEM((STG,), jnp.float32)]
        + [pltpu.VMEM((WIN, CW), jnp.float32) for _ in range(NBUF)]
        + [pltpu.VMEM((WIN,), jnp.int32) for _ in range(NBUF)]
        + [pltpu.VMEM((WIN,), jnp.int32) for _ in range(NBUF)]
        + [pltpu.SemaphoreType.DMA for _ in range(2 * NBUF)]
    )

    @functools.partial(
        pl.kernel,
        out_type=jax.ShapeDtypeStruct((NCHUNK * NP, CW), jnp.float32),
        mesh=mesh,
        scratch_types=scratch,
        compiler_params=pltpu.CompilerParams(use_tc_tiling_on_sc=False),
    )
    def spmm(*refs):
        row_hbm, col_hbm, val_hbm, src_hbm = refs[:4]
        out_hbm = refs[4]
        sc = refs[5:]
        acc = sc[0]
        rowbig, colbig, valbig = sc[1], sc[2], sc[3]
        gath = sc[4:4 + NBUF]
        coltmp = sc[4 + NBUF:4 + 2 * NBUF]
        rowtmp = sc[4 + 2 * NBUF:4 + 3 * NBUF]
        gsem = sc[4 + 3 * NBUF:4 + 3 * NBUF + NBUF]
        ssem = sc[4 + 3 * NBUF + NBUF:]

        cid = lax.axis_index("c")
        sid = lax.axis_index("s")
        e0 = sid * EPT
        r0 = sid * RPT

        def _pass(p, _):
            chunk = NCORE * p + cid
            coff = chunk * NP

            iblk = gath[0].at[pl.ds(0, IBK)]
            zv = jnp.zeros((16,), jnp.float32)

            def _zero(r, _):
                for u in range(CW // 16):
                    gath[0][r, pl.ds(u * 16, 16)] = zv
                return 0

            lax.fori_loop(0, IBK, _zero, 0)

            def _initz(b, _):
                pltpu.sync_copy(iblk, acc.at[pl.ds(r0 + b * IBK, IBK)])
                return 0

            lax.fori_loop(0, RPT // IBK, _initz, 0)
            plsc.subcore_barrier()

            def _win_idx(bj, w):
                o = w * WIN
                for u in range(WIN // 16):
                    s = pl.ds(u * 16, 16)
                    so = pl.ds(o + u * 16, 16)
                    coltmp[bj][s] = colbig[so] + coff
                    rowtmp[bj][s] = rowbig[so]

            def _gather(bj):
                pltpu.async_copy(src_hbm.at[coltmp[bj]], gath[bj], gsem[bj])

            def _wait_gather(bj):
                pltpu.make_async_copy(
                    src_hbm.at[coltmp[bj]], gath[bj], gsem[bj]).wait()

            def _scatter(bj):
                pltpu.async_copy(
                    gath[bj], acc.at[rowtmp[bj]], ssem[bj], add=True)

            def _wait_scatter(bj):
                pltpu.make_async_copy(
                    gath[bj], acc.at[rowtmp[bj]], ssem[bj]).wait()

            def _mul_win(bj, w):
                o = w * WIN

                @plsc.parallel_loop(0, WIN // 16, unroll=2)
                def _mul(k2):
                    vv = valbig[pl.ds(o + k2 * 16, 16)]
                    for t in range(16):
                        e = k2 * 16 + t
                        v = vv[t]
                        for u in range(CW // 16):
                            s = pl.ds(u * 16, 16)
                            gath[bj][e, s] = gath[bj][e, s] * v

            HB = NBUF // 2

            def _stage(stg, _):
                eb = e0 + stg * STG
                pltpu.sync_copy(row_hbm.at[pl.ds(eb, STG)], rowbig)
                pltpu.sync_copy(col_hbm.at[pl.ds(eb, STG)], colbig)
                pltpu.sync_copy(val_hbm.at[pl.ds(eb, STG)], valbig)

                for j in range(HB):
                    _win_idx(j, j)
                    _gather(j)

                def _step(k, _):
                    gA = 2 * k
                    gB = 2 * k + 1
                    for j in range(HB):
                        bj = HB + j

                        @pl.when(k > 0)
                        def _():
                            _wait_scatter(bj)
                        _win_idx(bj, gB * HB + j)
                        _gather(bj)
                    for j in range(HB):
                        _wait_gather(j)
                        _mul_win(j, gA * HB + j)
                        _scatter(j)
                    for j in range(HB):
                        @pl.when(k < NITER - 1)
                        def _():
                            _wait_scatter(j)
                            _win_idx(j, (2 * k + 2) * HB + j)
                            _gather(j)
                    for j in range(HB):
                        _wait_gather(HB + j)
                        _mul_win(HB + j, gB * HB + j)
                        _scatter(HB + j)
                    return 0

                lax.fori_loop(0, NITER, _step, 0)
                for j in range(NBUF):
                    _wait_scatter(j)
                return 0

            lax.fori_loop(0, NSTG, _stage, 0)
            plsc.subcore_barrier()

            pltpu.sync_copy(acc.at[pl.ds(r0, RPT)],
                            out_hbm.at[pl.ds(coff + r0, RPT)])
            plsc.subcore_barrier()
            return 0

        lax.fori_loop(0, NCHUNK // NCORE, _pass, 0)

    return spmm


_spmm = _make_spmm()

TN = 1000


def _proj_body(x0r, x1r, x2r, x3r, x4r, wr, br, outr):
    acc = jnp.zeros((TN, 512), jnp.float32)
    for m, xr in enumerate((x0r, x1r, x2r, x3r, x4r)):
        xcat = jnp.concatenate([xr[j] for j in range(NCHUNK)], axis=1)
        acc = acc + jnp.dot(xcat, wr[m], preferred_element_type=jnp.float32)
    outr[...] = acc + br[...]


def _project(xs_cm, w3, bias_t):
    in_specs = (
        [pl.BlockSpec((NCHUNK, TN, CW), lambda i: (0, i, 0)) for _ in range(5)]
        + [pl.BlockSpec((5, 512, 512), lambda i: (0, 0, 0)),
           pl.BlockSpec((1, 512), lambda i: (0, 0))]
    )
    return pl.pallas_call(
        _proj_body,
        grid=(N_NODES // TN,),
        in_specs=in_specs,
        out_specs=pl.BlockSpec((TN, 512), lambda i: (i, 0)),
        out_shape=jax.ShapeDtypeStruct((N_NODES, 512), jnp.float32),
    )(*xs_cm, w3, bias_t)


def _pad_edges(idx, val):
    pad = E_PAD - N_EDGES
    ar = jnp.arange(pad, dtype=jnp.int32)
    rows = jnp.concatenate([idx[0], ar % N_NODES])
    cols = jnp.concatenate([idx[1], (ar * 131) % N_NODES])
    vals = jnp.concatenate([val, jnp.zeros((pad,), jnp.float32)])
    return rows, cols, vals


def kernel(x, s0_idx, s0_val, s1_idx, s1_val, Theta, bias):
    b, n, c = x.shape
    x0 = x.reshape(n, c * b)
    x0c3 = jnp.pad(x0.reshape(n, NCHUNK, CW).transpose(1, 0, 2),
                   ((0, 0), (0, NP - n), (0, 0)))
    x0c = x0c3.reshape(NCHUNK * NP, CW)

    r0, c0, v0 = _pad_edges(s0_idx, s0_val)
    r1, c1, v1 = _pad_edges(s1_idx, s1_val)

    s1m = _spmm(r0, c0, v0, x0c)
    s2m = _spmm(r0, c0, v0, s1m)
    s3m = _spmm(r1, c1, v1, s1m)
    s4m = _spmm(r1, c1, v1, s3m)

    theta_r = Theta.reshape(128, 5, c)
    tm = jnp.transpose(theta_r, (1, 0, 2))
    tm_rep = jnp.repeat(tm, 4, axis=1)
    msk = jnp.tile(jnp.eye(4, dtype=jnp.float32), (128, 1))
    w2 = (tm_rep[:, :, None, :] * msk[None, :, :, None]).reshape(5, 512, 512)
    w2 = w2.at[0].add(-w2[2]).at[1].add(-w2[4])
    w2 = w2.at[2].multiply(2.0).at[4].multiply(2.0)
    bias_t = jnp.tile(bias, 4).reshape(1, 512)

    xs_cm = [a.reshape(NCHUNK, NP, CW) for a in (x0c, s1m, s2m, s3m, s4m)]
    out_cat = _project(xs_cm, w2, bias_t)
    return out_cat.reshape(n, 4, 128).transpose(1, 0, 2)


# speedup vs baseline: 1.0963x; 1.0623x over previous
import functools

import jax
import jax.numpy as jnp
from jax import lax
from jax.experimental import pallas as pl
from jax.experimental.pallas import tpu as pltpu
from jax.experimental.pallas import tpu_sc as plsc

N_NODES = 10000
NP = 10240
N_EDGES = 320000
CW = 128
NCHUNK = 4
NCORE = 2
NTILE = 16
EPT = 20480
E_PAD = EPT * NTILE
WIN = 128
NBUF = 2
STG = 2560
NITER = STG // (WIN * NBUF)
NSTG = EPT // STG
RPT = NP // NTILE
IBK = 128


def _make_spmm():
    mesh = plsc.VectorSubcoreMesh(core_axis_name="c", subcore_axis_name="s")

    scratch = (
        [pltpu.VMEM_SHARED((NP, CW), jnp.float32)]
        + [pltpu.VMEM((STG,), jnp.int32) for _ in range(2)]
        + [pltpu.VMEM((STG,), jnp.float32)]
        + [pltpu.VMEM((WIN, CW), jnp.float32) for _ in range(NBUF)]
        + [pltpu.VMEM((WIN,), jnp.int32) for _ in range(NBUF)]
        + [pltpu.VMEM((WIN,), jnp.int32) for _ in range(NBUF)]
        + [pltpu.SemaphoreType.DMA for _ in range(2 * NBUF)]
    )

    @functools.partial(
        pl.kernel,
        out_type=jax.ShapeDtypeStruct((NCHUNK * NP, CW), jnp.float32),
        mesh=mesh,
        scratch_types=scratch,
    )
    def spmm(*refs):
        row_hbm, col_hbm, val_hbm, src_hbm = refs[:4]
        out_hbm = refs[4]
        sc = refs[5:]
        acc = sc[0]
        rowbig, colbig, valbig = sc[1], sc[2], sc[3]
        gath = sc[4:4 + NBUF]
        coltmp = sc[4 + NBUF:4 + 2 * NBUF]
        rowtmp = sc[4 + 2 * NBUF:4 + 3 * NBUF]
        gsem = sc[4 + 3 * NBUF:4 + 3 * NBUF + NBUF]
        ssem = sc[4 + 3 * NBUF + NBUF:]

        cid = lax.axis_index("c")
        sid = lax.axis_index("s")
        e0 = sid * EPT
        r0 = sid * RPT

        def _pass(p, _):
            chunk = NCORE * p + cid
            coff = chunk * NP

            iblk = gath[0].at[pl.ds(0, IBK)]
            zv = jnp.zeros((16,), jnp.float32)

            def _zero(r, _):
                for u in range(CW // 16):
                    gath[0][r, pl.ds(u * 16, 16)] = zv
                return 0

            lax.fori_loop(0, IBK, _zero, 0)

            def _initz(b, _):
                pltpu.sync_copy(iblk, acc.at[pl.ds(r0 + b * IBK, IBK)])
                return 0

            lax.fori_loop(0, RPT // IBK, _initz, 0)
            plsc.subcore_barrier()

            def _win_idx(bj, w):
                o = w * WIN
                for u in range(WIN // 16):
                    s = pl.ds(u * 16, 16)
                    so = pl.ds(o + u * 16, 16)
                    coltmp[bj][s] = colbig[so] + coff
                    rowtmp[bj][s] = rowbig[so]

            def _gather(bj):
                pltpu.async_copy(src_hbm.at[coltmp[bj]], gath[bj], gsem[bj])

            def _wait_gather(bj):
                pltpu.make_async_copy(
                    src_hbm.at[coltmp[bj]], gath[bj], gsem[bj]).wait()

            def _scatter(bj):
                pltpu.async_copy(
                    gath[bj], acc.at[rowtmp[bj]], ssem[bj], add=True)

            def _wait_scatter(bj):
                pltpu.make_async_copy(
                    gath[bj], acc.at[rowtmp[bj]], ssem[bj]).wait()

            def _mul_win(bj, w):
                o = w * WIN

                @plsc.parallel_loop(0, WIN // 16, unroll=2)
                def _mul(k2):
                    vv = valbig[pl.ds(o + k2 * 16, 16)]
                    for t in range(16):
                        e = k2 * 16 + t
                        v = vv[t]
                        for u in range(CW // 16):
                            s = pl.ds(u * 16, 16)
                            gath[bj][e, s] = gath[bj][e, s] * v

            HB = NBUF // 2

            def _stage(stg, _):
                eb = e0 + stg * STG
                pltpu.sync_copy(row_hbm.at[pl.ds(eb, STG)], rowbig)
                pltpu.sync_copy(col_hbm.at[pl.ds(eb, STG)], colbig)
                pltpu.sync_copy(val_hbm.at[pl.ds(eb, STG)], valbig)

                for j in range(HB):
                    _win_idx(j, j)
                    _gather(j)

                def _step(k, _):
                    gA = 2 * k
                    gB = 2 * k + 1
                    for j in range(HB):
                        bj = HB + j

                        @pl.when(k > 0)
                        def _():
                            _wait_scatter(bj)
                        _win_idx(bj, gB * HB + j)
                        _gather(bj)
                    for j in range(HB):
                        _wait_gather(j)
                        _mul_win(j, gA * HB + j)
                        _scatter(j)
                    for j in range(HB):
                        @pl.when(k < NITER - 1)
                        def _():
                            _wait_scatter(j)
                            _win_idx(j, (2 * k + 2) * HB + j)
                            _gather(j)
                    for j in range(HB):
                        _wait_gather(HB + j)
                        _mul_win(HB + j, gB * HB + j)
                        _scatter(HB + j)
                    return 0

                lax.fori_loop(0, NITER, _step, 0)
                for j in range(NBUF):
                    _wait_scatter(j)
                return 0

            lax.fori_loop(0, NSTG, _stage, 0)
            plsc.subcore_barrier()

            pltpu.sync_copy(acc.at[pl.ds(r0, RPT)],
                            out_hbm.at[pl.ds(coff + r0, RPT)])
            plsc.subcore_barrier()
            return 0

        lax.fori_loop(0, NCHUNK // NCORE, _pass, 0)

    return spmm


_spmm = _make_spmm()

TN = 1000


def _proj_body(x0r, x1r, x2r, x3r, x4r, wr, br, outr):
    acc = jnp.zeros((TN, 512), jnp.float32)
    for m, xr in enumerate((x0r, x1r, x2r, x3r, x4r)):
        xcat = jnp.concatenate([xr[j] for j in range(NCHUNK)], axis=1)
        acc = acc + jnp.dot(xcat, wr[m], preferred_element_type=jnp.float32)
    outr[...] = acc + br[...]


def _project(xs_cm, w3, bias_t):
    in_specs = (
        [pl.BlockSpec((NCHUNK, TN, CW), lambda i: (0, i, 0)) for _ in range(5)]
        + [pl.BlockSpec((5, 512, 512), lambda i: (0, 0, 0)),
           pl.BlockSpec((1, 512), lambda i: (0, 0))]
    )
    return pl.pallas_call(
        _proj_body,
        grid=(N_NODES // TN,),
        in_specs=in_specs,
        out_specs=pl.BlockSpec((TN, 512), lambda i: (i, 0)),
        out_shape=jax.ShapeDtypeStruct((N_NODES, 512), jnp.float32),
    )(*xs_cm, w3, bias_t)


def _pad_edges(idx, val):
    pad = E_PAD - N_EDGES
    ar = jnp.arange(pad, dtype=jnp.int32)
    rows = jnp.concatenate([idx[0], ar % N_NODES])
    cols = jnp.concatenate([idx[1], (ar * 131) % N_NODES])
    vals = jnp.concatenate([val, jnp.zeros((pad,), jnp.float32)])
    return rows, cols, vals


def kernel(x, s0_idx, s0_val, s1_idx, s1_val, Theta, bias):
    b, n, c = x.shape
    x0 = x.reshape(n, c * b)
    x0c3 = jnp.pad(x0.reshape(n, NCHUNK, CW).transpose(1, 0, 2),
                   ((0, 0), (0, NP - n), (0, 0)))
    x0c = x0c3.reshape(NCHUNK * NP, CW)

    r0, c0, v0 = _pad_edges(s0_idx, s0_val)
    r1, c1, v1 = _pad_edges(s1_idx, s1_val)

    s1m = _spmm(r0, c0, v0, x0c)
    s2m = _spmm(r0, c0, v0, s1m)
    s3m = _spmm(r1, c1, v1, s1m)
    s4m = _spmm(r1, c1, v1, s3m)

    theta_r = Theta.reshape(128, 5, c)
    tm = jnp.transpose(theta_r, (1, 0, 2))
    tm_rep = jnp.repeat(tm, 4, axis=1)
    msk = jnp.tile(jnp.eye(4, dtype=jnp.float32), (128, 1))
    w2 = (tm_rep[:, :, None, :] * msk[None, :, :, None]).reshape(5, 512, 512)
    w2 = w2.at[0].add(-w2[2]).at[1].add(-w2[4])
    w2 = w2.at[2].multiply(2.0).at[4].multiply(2.0)
    bias_t = jnp.tile(bias, 4).reshape(1, 512)

    xs_cm = [a.reshape(NCHUNK, NP, CW) for a in (x0c, s1m, s2m, s3m, s4m)]
    out_cat = _project(xs_cm, w2, bias_t)
    return out_cat.reshape(n, 4, 128).transpose(1, 0, 2)
